# baseline (speedup 1.0000x reference)
import functools

import jax
import jax.numpy as jnp
from jax import lax
from jax.experimental import pallas as pl
from jax.experimental.pallas import tpu as pltpu
from jax.experimental.pallas import tpu_sc as plsc

N = 10000
NPAD = 10240
E = 320000
D = 128
DH = D // 2
G = 64
BN_EPS = 1e-5

NC = 2
NS = 16
C = 128
EPT = 20480
EPAD = EPT * NS
NCH = EPT // C
NB = 5
KG = 3
NHB = 2
RPT = NPAD // NS


def _sc_agg_body(h_hbm, src_hbm, dst_hbm, out_hbm, *rest):
    gbuf = rest[:NB]
    sbufs = rest[NB:2 * NB]
    dbufs = rest[2 * NB:3 * NB]
    acc = rest[3 * NB]
    h_spm = rest[3 * NB + 1]
    gsem = rest[3 * NB + 2:3 * NB + 2 + NB]
    ssem = rest[3 * NB + 2 + NB:3 * NB + 2 + 2 * NB]
    isem = rest[3 * NB + 2 + 2 * NB:]
    cid = lax.axis_index("c")
    sid = lax.axis_index("s")
    row0 = sid * RPT
    hview = h_hbm.at[cid]
    sview = src_hbm.at[sid]
    dview = dst_hbm.at[sid]

    pltpu.sync_copy(hview.at[pl.ds(row0, RPT)], acc.at[pl.ds(row0, RPT)])
    pltpu.sync_copy(hview.at[pl.ds(row0, RPT)], h_spm.at[pl.ds(row0, RPT)])
    plsc.subcore_barrier()

    def fire_i(j, b):
        pltpu.async_copy(sview.at[j], sbufs[b], isem[b])
        pltpu.async_copy(dview.at[j], dbufs[b], isem[b])

    def wait_i(b):
        pltpu.make_async_copy(sview.at[0], sbufs[b], isem[b]).wait()
        pltpu.make_async_copy(dview.at[0], dbufs[b], isem[b]).wait()

    def fire_g(j, b):
        if b < NHB:
            pltpu.async_copy(hview.at[sbufs[b]], gbuf[b], gsem[b])
        else:
            pltpu.async_copy(h_spm.at[sbufs[b]], gbuf[b], gsem[b])

    def wait_g(b):
        if b < NHB:
            pltpu.make_async_copy(hview.at[sbufs[0]], gbuf[b], gsem[b]).wait()
        else:
            pltpu.make_async_copy(h_spm.at[sbufs[0]], gbuf[b], gsem[b]).wait()

    def fire_s(j, b):
        pltpu.async_copy(gbuf[b], acc.at[dbufs[b]], ssem[b], add=True)

    def wait_s(b):
        pltpu.make_async_copy(gbuf[b], acc.at[dbufs[0]], ssem[b]).wait()

    for t in range(KG):
        fire_i(t, t)
    for t in range(KG - 1):
        wait_i(t)
        fire_g(t, t)

    def body(it, carry):
        j0 = it * NB
        for b in range(NB):
            j = j0 + b
            wait_g(b)
            fire_s(j, b)
            bk = (b + KG) % NB
            bk1 = (b + KG - 1) % NB

            @pl.when(j >= NB - KG)
            def _():
                wait_s(bk)

            @pl.when(j + KG < NCH)
            def _():
                fire_i(j + KG, bk)

            @pl.when(j + KG - 1 < NCH)
            def _():
                wait_i(bk1)
                fire_g(j + KG - 1, bk1)

        return carry

    lax.fori_loop(0, NCH // NB, body, 0)
    for j in range(NCH - (NB - KG), NCH):
        wait_s(j % NB)
    plsc.subcore_barrier()

    pltpu.sync_copy(acc.at[pl.ds(row0, RPT)], out_hbm.at[cid].at[pl.ds(row0, RPT)])


@functools.cache
def _get_sc_agg():
    mesh = plsc.VectorSubcoreMesh(core_axis_name="c", subcore_axis_name="s")
    return pl.kernel(
        _sc_agg_body,
        out_type=jax.ShapeDtypeStruct((NC, NPAD, DH), jnp.float32),
        mesh=mesh,
        scratch_types=[pltpu.VMEM((C, DH), jnp.float32) for _ in range(NB)]
        + [pltpu.VMEM((C,), jnp.int32) for _ in range(NB)]
        + [pltpu.VMEM((C,), jnp.int32) for _ in range(NB)]
        + [
            pltpu.VMEM_SHARED((NPAD, DH), jnp.float32),
            pltpu.VMEM_SHARED((NPAD, DH), jnp.float32),
        ]
        + [pltpu.SemaphoreType.DMA for _ in range(3 * NB)],
        compiler_params=pltpu.CompilerParams(use_tc_tiling_on_sc=False),
    )


def _mlp_core(p, w1, b1, w2, b2, gam, bet):
    z = jnp.concatenate([p[0, 0:N, :], p[1, 0:N, :]], axis=1)
    a1 = jnp.maximum(lax.dot(z, w1[...], preferred_element_type=jnp.float32) + b1[...], 0.0)
    a2 = jnp.maximum(lax.dot(a1, w2[...], preferred_element_type=jnp.float32) + b2[...], 0.0)
    m = jnp.mean(a2, axis=0, keepdims=True)
    c = a2 - m
    v = jnp.mean(c * c, axis=0, keepdims=True)
    return c * lax.rsqrt(v + BN_EPS) * gam[...] + bet[...]


def _mlp_bn_body(p, w1, b1, w2, b2, gam, bet, out):
    hn = _mlp_core(p, w1, b1, w2, b2, gam, bet)
    out[0, 0:N, :] = hn[:, 0:DH]
    out[1, 0:N, :] = hn[:, DH:D]


def _mlp_bn_pool_body(p, w1, b1, w2, b2, gam, bet, bat, hn_out, hg_out):
    hn = _mlp_core(p, w1, b1, w2, b2, gam, bet)
    hn_out[...] = hn
    onehot = (bat[...] == lax.broadcasted_iota(jnp.int32, (1, G), 1)).astype(jnp.float32)
    counts = jnp.sum(onehot, axis=0, keepdims=True)
    recip = 1.0 / jnp.maximum(counts, 1.0)
    hg_out[...] = lax.dot_general(
        onehot * recip, hn, (((0,), (0,)), ((), ())),
        preferred_element_type=jnp.float32,
    )


_mlp_bn = pl.pallas_call(
    _mlp_bn_body,
    out_shape=jax.ShapeDtypeStruct((NC, NPAD, DH), jnp.float32),
)

_mlp_bn_pool = pl.pallas_call(
    _mlp_bn_pool_body,
    out_shape=(
        jax.ShapeDtypeStruct((N, D), jnp.float32),
        jax.ShapeDtypeStruct((G, D), jnp.float32),
    ),
)


def kernel(x, edge_index, edge_weight, batch, params):
    src = jnp.pad(edge_index[0], (0, EPAD - E)).reshape(NS, NCH, C)
    dst = jnp.pad(edge_index[1], (0, EPAD - E), constant_values=N).reshape(NS, NCH, C)
    bat2d = batch.reshape(N, 1)
    xp = jnp.pad(x, ((0, NPAD - N), (0, 0)))
    h2 = jnp.stack([xp[:, 0:DH], xp[:, DH:D]])
    n_layers = len(params)
    h_graph = None
    h_node = None
    sc_agg = _get_sc_agg()
    for i, p in enumerate(params):
        parts = sc_agg(h2, src, dst)
        args = (
            parts,
            p["W1"], p["b1"].reshape(1, D),
            p["W2"], p["b2"].reshape(1, D),
            p["gamma"].reshape(1, D), p["beta"].reshape(1, D),
        )
        if i < n_layers - 1:
            h2 = _mlp_bn(*args)
        else:
            h_node, h_graph = _mlp_bn_pool(*args, bat2d)
    return (h_node, h_graph)


# speedup vs baseline: 1.1203x; 1.1203x over previous
import functools

import jax
import jax.numpy as jnp
from jax import lax
from jax.experimental import pallas as pl
from jax.experimental.pallas import tpu as pltpu
from jax.experimental.pallas import tpu_sc as plsc

N = 10000
NPAD = 10240
E = 320000
D = 128
DH = D // 2
G = 64
BN_EPS = 1e-5

NC = 2
NS = 16
C = 128
EPT = 20480
EPAD = EPT * NS
NCH = EPT // C
NB = 5
KG = 3
NHB = 1
RPT = NPAD // NS


def _sc_agg_body(h_hbm, src_hbm, dst_hbm, out_hbm, *rest):
    gbuf = rest[:NB]
    sbufs = rest[NB:2 * NB]
    dbufs = rest[2 * NB:3 * NB]
    acc = rest[3 * NB]
    h_spm = rest[3 * NB + 1]
    gsem = rest[3 * NB + 2:3 * NB + 2 + NB]
    ssem = rest[3 * NB + 2 + NB:3 * NB + 2 + 2 * NB]
    isem = rest[3 * NB + 2 + 2 * NB:]
    cid = lax.axis_index("c")
    sid = lax.axis_index("s")
    row0 = sid * RPT
    hview = h_hbm.at[cid]
    sview = src_hbm.at[sid]
    dview = dst_hbm.at[sid]

    pltpu.sync_copy(hview.at[pl.ds(row0, RPT)], acc.at[pl.ds(row0, RPT)])
    pltpu.sync_copy(hview.at[pl.ds(row0, RPT)], h_spm.at[pl.ds(row0, RPT)])
    plsc.subcore_barrier()

    def fire_i(j, b):
        pltpu.async_copy(sview.at[j], sbufs[b], isem[b])
        pltpu.async_copy(dview.at[j], dbufs[b], isem[b])

    def wait_i(b):
        pltpu.make_async_copy(sview.at[0], sbufs[b], isem[b]).wait()
        pltpu.make_async_copy(dview.at[0], dbufs[b], isem[b]).wait()

    def fire_g(j, b):
        if b < NHB:
            pltpu.async_copy(hview.at[sbufs[b]], gbuf[b], gsem[b])
        else:
            pltpu.async_copy(h_spm.at[sbufs[b]], gbuf[b], gsem[b])

    def wait_g(b):
        if b < NHB:
            pltpu.make_async_copy(hview.at[sbufs[0]], gbuf[b], gsem[b]).wait()
        else:
            pltpu.make_async_copy(h_spm.at[sbufs[0]], gbuf[b], gsem[b]).wait()

    def fire_s(j, b):
        pltpu.async_copy(gbuf[b], acc.at[dbufs[b]], ssem[b], add=True)

    def wait_s(b):
        pltpu.make_async_copy(gbuf[b], acc.at[dbufs[0]], ssem[b]).wait()

    for t in range(KG):
        fire_i(t, t)
    for t in range(KG - 1):
        wait_i(t)
        fire_g(t, t)

    def body(it, carry):
        j0 = it * NB
        for b in range(NB):
            j = j0 + b
            wait_g(b)
            fire_s(j, b)
            bk = (b + KG) % NB
            bk1 = (b + KG - 1) % NB

            @pl.when(j >= NB - KG)
            def _():
                wait_s(bk)

            @pl.when(j + KG < NCH)
            def _():
                fire_i(j + KG, bk)

            @pl.when(j + KG - 1 < NCH)
            def _():
                wait_i(bk1)
                fire_g(j + KG - 1, bk1)

        return carry

    lax.fori_loop(0, NCH // NB, body, 0)
    for j in range(NCH - (NB - KG), NCH):
        wait_s(j % NB)
    plsc.subcore_barrier()

    pltpu.sync_copy(acc.at[pl.ds(row0, RPT)], out_hbm.at[cid].at[pl.ds(row0, RPT)])


@functools.cache
def _get_sc_agg():
    mesh = plsc.VectorSubcoreMesh(core_axis_name="c", subcore_axis_name="s")
    return pl.kernel(
        _sc_agg_body,
        out_type=jax.ShapeDtypeStruct((NC, NPAD, DH), jnp.float32),
        mesh=mesh,
        scratch_types=[pltpu.VMEM((C, DH), jnp.float32) for _ in range(NB)]
        + [pltpu.VMEM((C,), jnp.int32) for _ in range(NB)]
        + [pltpu.VMEM((C,), jnp.int32) for _ in range(NB)]
        + [
            pltpu.VMEM_SHARED((NPAD, DH), jnp.float32),
            pltpu.VMEM_SHARED((NPAD, DH), jnp.float32),
        ]
        + [pltpu.SemaphoreType.DMA for _ in range(3 * NB)],
        compiler_params=pltpu.CompilerParams(use_tc_tiling_on_sc=False),
    )


def _mlp_core(p, w1, b1, w2, b2, gam, bet):
    z = jnp.concatenate([p[0, 0:N, :], p[1, 0:N, :]], axis=1)
    a1 = jnp.maximum(lax.dot(z, w1[...], preferred_element_type=jnp.float32) + b1[...], 0.0)
    a2 = jnp.maximum(lax.dot(a1, w2[...], preferred_element_type=jnp.float32) + b2[...], 0.0)
    m = jnp.mean(a2, axis=0, keepdims=True)
    c = a2 - m
    v = jnp.mean(c * c, axis=0, keepdims=True)
    return c * lax.rsqrt(v + BN_EPS) * gam[...] + bet[...]


def _mlp_bn_body(p, w1, b1, w2, b2, gam, bet, out):
    hn = _mlp_core(p, w1, b1, w2, b2, gam, bet)
    out[0, 0:N, :] = hn[:, 0:DH]
    out[1, 0:N, :] = hn[:, DH:D]


def _mlp_bn_pool_body(p, w1, b1, w2, b2, gam, bet, bat, hn_out, hg_out):
    hn = _mlp_core(p, w1, b1, w2, b2, gam, bet)
    hn_out[...] = hn
    onehot = (bat[...] == lax.broadcasted_iota(jnp.int32, (1, G), 1)).astype(jnp.float32)
    counts = jnp.sum(onehot, axis=0, keepdims=True)
    recip = 1.0 / jnp.maximum(counts, 1.0)
    hg_out[...] = lax.dot_general(
        onehot * recip, hn, (((0,), (0,)), ((), ())),
        preferred_element_type=jnp.float32,
    )


_mlp_bn = pl.pallas_call(
    _mlp_bn_body,
    out_shape=jax.ShapeDtypeStruct((NC, NPAD, DH), jnp.float32),
)

_mlp_bn_pool = pl.pallas_call(
    _mlp_bn_pool_body,
    out_shape=(
        jax.ShapeDtypeStruct((N, D), jnp.float32),
        jax.ShapeDtypeStruct((G, D), jnp.float32),
    ),
)


def kernel(x, edge_index, edge_weight, batch, params):
    src = jnp.pad(edge_index[0], (0, EPAD - E)).reshape(NS, NCH, C)
    dst = jnp.pad(edge_index[1], (0, EPAD - E), constant_values=N).reshape(NS, NCH, C)
    bat2d = batch.reshape(N, 1)
    xp = jnp.pad(x, ((0, NPAD - N), (0, 0)))
    h2 = jnp.stack([xp[:, 0:DH], xp[:, DH:D]])
    n_layers = len(params)
    h_graph = None
    h_node = None
    sc_agg = _get_sc_agg()
    for i, p in enumerate(params):
        parts = sc_agg(h2, src, dst)
        args = (
            parts,
            p["W1"], p["b1"].reshape(1, D),
            p["W2"], p["b2"].reshape(1, D),
            p["gamma"].reshape(1, D), p["beta"].reshape(1, D),
        )
        if i < n_layers - 1:
            h2 = _mlp_bn(*args)
        else:
            h_node, h_graph = _mlp_bn_pool(*args, bat2d)
    return (h_node, h_graph)


# speedup vs baseline: 1.4512x; 1.2954x over previous
import functools

import jax
import jax.numpy as jnp
from jax import lax
from jax.experimental import pallas as pl
from jax.experimental.pallas import tpu as pltpu
from jax.experimental.pallas import tpu_sc as plsc

N = 10000
NPAD = 10240
E = 320000
D = 128
DH = D // 2
G = 64
BN_EPS = 1e-5

NC = 2
NS = 16
C = 128
EPT = 20480
EPAD = EPT * NS
NCH = EPT // C
NB = 5
KG = 3
RPT = NPAD // NS


def _sc_agg_body(h_hbm, src_hbm, dst_hbm, out_hbm, *rest):
    gbuf = rest[:NB]
    sbufs = rest[NB:2 * NB]
    dbufs = rest[2 * NB:3 * NB]
    acc = rest[3 * NB]
    h_spm = rest[3 * NB + 1]
    gsem = rest[3 * NB + 2:3 * NB + 2 + NB]
    ssem = rest[3 * NB + 2 + NB:3 * NB + 2 + 2 * NB]
    isem = rest[3 * NB + 2 + 2 * NB:]
    cid = lax.axis_index("c")
    sid = lax.axis_index("s")
    row0 = sid * RPT
    hview = h_hbm.at[cid]
    sview = src_hbm.at[sid]
    dview = dst_hbm.at[sid]

    pltpu.sync_copy(hview.at[pl.ds(row0, RPT)], acc.at[pl.ds(row0, RPT)])
    pltpu.sync_copy(hview.at[pl.ds(row0, RPT)], h_spm.at[pl.ds(row0, RPT)])
    plsc.subcore_barrier()

    def fire_i(j, b):
        pltpu.async_copy(sview.at[j], sbufs[b], isem[b])
        pltpu.async_copy(dview.at[j], dbufs[b], isem[b])

    def wait_i(b):
        pltpu.make_async_copy(sview.at[0], sbufs[b], isem[b]).wait()
        pltpu.make_async_copy(dview.at[0], dbufs[b], isem[b]).wait()

    def fire_g(j, b):
        pltpu.async_copy(h_spm.at[sbufs[b]], gbuf[b], gsem[b])

    def wait_g(b):
        pltpu.make_async_copy(h_spm.at[sbufs[0]], gbuf[b], gsem[b]).wait()

    def fire_s(j, b):
        pltpu.async_copy(gbuf[b], acc.at[dbufs[b]], ssem[b], add=True)

    def wait_s(b):
        pltpu.make_async_copy(gbuf[b], acc.at[dbufs[0]], ssem[b]).wait()

    for t in range(KG):
        fire_i(t, t)
    for t in range(KG - 1):
        wait_i(t)
        fire_g(t, t)

    def body(it, carry):
        j0 = it * NB
        for b in range(NB):
            j = j0 + b
            wait_g(b)
            fire_s(j, b)
            bk = (b + KG) % NB
            bk1 = (b + KG - 1) % NB

            @pl.when(j >= NB - KG)
            def _():
                wait_s(bk)

            @pl.when(j + KG < NCH)
            def _():
                fire_i(j + KG, bk)

            @pl.when(j + KG - 1 < NCH)
            def _():
                wait_i(bk1)
                fire_g(j + KG - 1, bk1)

        return carry

    lax.fori_loop(0, NCH // NB, body, 0)
    for j in range(NCH - (NB - KG), NCH):
        wait_s(j % NB)
    plsc.subcore_barrier()

    pltpu.sync_copy(acc.at[pl.ds(row0, RPT)], out_hbm.at[cid].at[pl.ds(row0, RPT)])


@functools.cache
def _get_sc_agg():
    mesh = plsc.VectorSubcoreMesh(core_axis_name="c", subcore_axis_name="s")
    return pl.kernel(
        _sc_agg_body,
        out_type=jax.ShapeDtypeStruct((NC, NPAD, DH), jnp.float32),
        mesh=mesh,
        scratch_types=[pltpu.VMEM((C, DH), jnp.float32) for _ in range(NB)]
        + [pltpu.VMEM((C,), jnp.int32) for _ in range(NB)]
        + [pltpu.VMEM((C,), jnp.int32) for _ in range(NB)]
        + [
            pltpu.VMEM_SHARED((NPAD, DH), jnp.float32),
            pltpu.VMEM_SHARED((NPAD, DH), jnp.float32),
        ]
        + [pltpu.SemaphoreType.DMA for _ in range(3 * NB)],
        compiler_params=pltpu.CompilerParams(use_tc_tiling_on_sc=False),
    )


def _mlp_core(p, w1, b1, w2, b2, gam, bet):
    z = jnp.concatenate([p[0, 0:N, :], p[1, 0:N, :]], axis=1)
    a1 = jnp.maximum(lax.dot(z, w1[...], preferred_element_type=jnp.float32) + b1[...], 0.0)
    a2 = jnp.maximum(lax.dot(a1, w2[...], preferred_element_type=jnp.float32) + b2[...], 0.0)
    m = jnp.mean(a2, axis=0, keepdims=True)
    c = a2 - m
    v = jnp.mean(c * c, axis=0, keepdims=True)
    return c * lax.rsqrt(v + BN_EPS) * gam[...] + bet[...]


def _mlp_bn_body(p, w1, b1, w2, b2, gam, bet, out):
    hn = _mlp_core(p, w1, b1, w2, b2, gam, bet)
    out[0, 0:N, :] = hn[:, 0:DH]
    out[1, 0:N, :] = hn[:, DH:D]


def _mlp_bn_pool_body(p, w1, b1, w2, b2, gam, bet, bat, hn_out, hg_out):
    hn = _mlp_core(p, w1, b1, w2, b2, gam, bet)
    hn_out[...] = hn
    onehot = (bat[...] == lax.broadcasted_iota(jnp.int32, (1, G), 1)).astype(jnp.float32)
    counts = jnp.sum(onehot, axis=0, keepdims=True)
    recip = 1.0 / jnp.maximum(counts, 1.0)
    hg_out[...] = lax.dot_general(
        onehot * recip, hn, (((0,), (0,)), ((), ())),
        preferred_element_type=jnp.float32,
    )


_mlp_bn = pl.pallas_call(
    _mlp_bn_body,
    out_shape=jax.ShapeDtypeStruct((NC, NPAD, DH), jnp.float32),
)

_mlp_bn_pool = pl.pallas_call(
    _mlp_bn_pool_body,
    out_shape=(
        jax.ShapeDtypeStruct((N, D), jnp.float32),
        jax.ShapeDtypeStruct((G, D), jnp.float32),
    ),
)


def kernel(x, edge_index, edge_weight, batch, params):
    src = jnp.pad(edge_index[0], (0, EPAD - E)).reshape(NS, NCH, C)
    dst = jnp.pad(edge_index[1], (0, EPAD - E), constant_values=N).reshape(NS, NCH, C)
    bat2d = batch.reshape(N, 1)
    xp = jnp.pad(x, ((0, NPAD - N), (0, 0)))
    h2 = jnp.stack([xp[:, 0:DH], xp[:, DH:D]])
    n_layers = len(params)
    h_graph = None
    h_node = None
    sc_agg = _get_sc_agg()
    for i, p in enumerate(params):
        parts = sc_agg(h2, src, dst)
        args = (
            parts,
            p["W1"], p["b1"].reshape(1, D),
            p["W2"], p["b2"].reshape(1, D),
            p["gamma"].reshape(1, D), p["beta"].reshape(1, D),
        )
        if i < n_layers - 1:
            h2 = _mlp_bn(*args)
        else:
            h_node, h_graph = _mlp_bn_pool(*args, bat2d)
    return (h_node, h_graph)
